# Initial kernel scaffold; baseline (speedup 1.0000x reference)
import jax
import jax.numpy as jnp
from jax.experimental import pallas as pl


def kernel(predictions):
    raise NotImplementedError("write your pallas kernel here")


# speedup vs baseline: 11.8153x; 11.8153x over previous
import jax
import jax.numpy as jnp
from jax.experimental import pallas as pl
from jax.experimental.pallas import tpu as pltpu

_CONF = 0.25
_IOU = 0.7
_MAXDET = 300
_NCLS = 80
_N = 20000
_LANES = 128
_ROWS = 160
_NPAD = _ROWS * _LANES
_NEG = -1e30


def _nms_kernel(pred_ref, out_ref):
    x = pred_ref[0, 0]
    y = pred_ref[0, 1]
    w = pred_ref[0, 2] * 0.5
    h = pred_ref[0, 3] * 0.5
    x1 = x - w
    y1 = y - h
    x2 = x + w
    y2 = y + h

    s = pred_ref[0, 4]
    c = jnp.zeros_like(s)
    for i in range(1, _NCLS):
        v = pred_ref[0, 4 + i]
        take = v > s
        c = jnp.where(take, float(i), c)
        s = jnp.maximum(s, v)

    ms0 = jnp.where(s > _CONF, s, _NEG)
    rr = jax.lax.broadcasted_iota(jnp.int32, (_ROWS, _LANES), 0)
    ll = jax.lax.broadcasted_iota(jnp.int32, (_ROWS, _LANES), 1)
    ii = rr * _LANES + ll
    area = (x2 - x1) * (y2 - y1)

    def body(i, ms):
        m = jnp.max(ms)
        has = m > (_NEG * 0.5)
        eq = ms == m
        idx = jnp.min(jnp.where(eq, ii, jnp.int32(2 ** 30)))
        pick = ii == idx
        bx1 = jnp.sum(jnp.where(pick, x1, 0.0))
        by1 = jnp.sum(jnp.where(pick, y1, 0.0))
        bx2 = jnp.sum(jnp.where(pick, x2, 0.0))
        by2 = jnp.sum(jnp.where(pick, y2, 0.0))
        bc = jnp.sum(jnp.where(pick, c, 0.0))
        ix1 = jnp.maximum(bx1, x1)
        iy1 = jnp.maximum(by1, y1)
        ix2 = jnp.minimum(bx2, x2)
        iy2 = jnp.minimum(by2, y2)
        inter = jnp.maximum(ix2 - ix1, 0.0) * jnp.maximum(iy2 - iy1, 0.0)
        a1 = (bx2 - bx1) * (by2 - by1)
        iou = inter / (a1 + area - inter + 1e-07)
        kill = jnp.logical_and(jnp.logical_or(iou > _IOU, pick), has)
        ms = jnp.where(kill, _NEG, ms)

        valid = jnp.where(has, 1.0, 0.0)
        li = jax.lax.broadcasted_iota(jnp.int32, (1, _LANES), 1)
        row = jnp.where(
            li == 0, bx1,
            jnp.where(li == 1, by1,
                      jnp.where(li == 2, bx2,
                                jnp.where(li == 3, by2,
                                          jnp.where(li == 4, m,
                                                    jnp.where(li == 5, bc, 0.0))))))
        out_ref[0, pl.ds(i, 1), :] = row * valid
        return ms

    jax.lax.fori_loop(0, _MAXDET, body, ms0)


def kernel(predictions):
    b = predictions.shape[0]
    pred = jnp.pad(predictions, ((0, 0), (0, _NPAD - _N), (0, 0)))
    pred = pred.transpose(0, 2, 1).reshape(b, 4 + _NCLS, _ROWS, _LANES)
    out = pl.pallas_call(
        _nms_kernel,
        grid=(b,),
        in_specs=[pl.BlockSpec((1, 4 + _NCLS, _ROWS, _LANES),
                               lambda i: (i, 0, 0, 0))],
        out_specs=pl.BlockSpec((1, _MAXDET, _LANES), lambda i: (i, 0, 0)),
        out_shape=jax.ShapeDtypeStruct((b, _MAXDET, _LANES), jnp.float32),
    )(pred)
    return out[:, :, :6]


# speedup vs baseline: 14.6490x; 1.2398x over previous
import jax
import jax.numpy as jnp
from jax.experimental import pallas as pl
from jax.experimental.pallas import tpu as pltpu

_CONF = 0.25
_IOU = 0.7
_MAXDET = 300
_NCLS = 80
_N = 20000
_LANES = 128
_ROWS = 160
_NPAD = _ROWS * _LANES
_NEG = -1e30
_B = 4


def _nms_kernel(pred_ref, out_ref, x1_ref, y1_ref, x2_ref, y2_ref, c_ref,
                area_ref):
    ms_init = []
    for b in range(_B):
        x = pred_ref[b, 0]
        y = pred_ref[b, 1]
        w = pred_ref[b, 2] * 0.5
        h = pred_ref[b, 3] * 0.5
        x1 = x - w
        y1 = y - h
        x2 = x + w
        y2 = y + h
        s = pred_ref[b, 4]
        c = jnp.zeros_like(s)
        for i in range(1, _NCLS):
            v = pred_ref[b, 4 + i]
            c = jnp.where(v > s, float(i), c)
            s = jnp.maximum(s, v)
        x1_ref[b] = x1
        y1_ref[b] = y1
        x2_ref[b] = x2
        y2_ref[b] = y2
        c_ref[b] = c
        area_ref[b] = (x2 - x1) * (y2 - y1)
        ms_init.append(jnp.where(s > _CONF, s, _NEG))

    rr = jax.lax.broadcasted_iota(jnp.int32, (_ROWS, _LANES), 0)
    ll = jax.lax.broadcasted_iota(jnp.int32, (_ROWS, _LANES), 1)
    ii = rr * _LANES + ll
    lane1 = jax.lax.broadcasted_iota(jnp.int32, (1, _LANES), 1)

    def body(i, carry):
        new = []
        for b in range(_B):
            ms = carry[b]
            m = jnp.max(ms)
            has = m > (_NEG * 0.5)
            idx = jnp.min(jnp.where(ms == m, ii, jnp.int32(2 ** 30)))
            r = idx // _LANES
            l = idx - r * _LANES
            lm = lane1 == l
            bx1 = jnp.sum(jnp.where(lm, x1_ref[b, pl.ds(r, 1), :], 0.0))
            by1 = jnp.sum(jnp.where(lm, y1_ref[b, pl.ds(r, 1), :], 0.0))
            bx2 = jnp.sum(jnp.where(lm, x2_ref[b, pl.ds(r, 1), :], 0.0))
            by2 = jnp.sum(jnp.where(lm, y2_ref[b, pl.ds(r, 1), :], 0.0))
            bc = jnp.sum(jnp.where(lm, c_ref[b, pl.ds(r, 1), :], 0.0))

            x1 = x1_ref[b]
            y1 = y1_ref[b]
            x2 = x2_ref[b]
            y2 = y2_ref[b]
            inter = (jnp.maximum(jnp.minimum(bx2, x2) - jnp.maximum(bx1, x1), 0.0)
                     * jnp.maximum(jnp.minimum(by2, y2) - jnp.maximum(by1, y1), 0.0))
            a1 = (bx2 - bx1) * (by2 - by1)
            iou = inter / (a1 + area_ref[b] - inter + 1e-07)
            kill = jnp.logical_and(
                jnp.logical_or(iou > _IOU, ii == idx), has)
            new.append(jnp.where(kill, _NEG, ms))

            valid = jnp.where(has, 1.0, 0.0)
            row = jnp.where(
                lane1 == 0, bx1,
                jnp.where(lane1 == 1, by1,
                          jnp.where(lane1 == 2, bx2,
                                    jnp.where(lane1 == 3, by2,
                                              jnp.where(lane1 == 4, m,
                                                        jnp.where(lane1 == 5, bc,
                                                                  0.0))))))
            out_ref[b, pl.ds(i, 1), :] = row * valid
        return tuple(new)

    jax.lax.fori_loop(0, _MAXDET, body, tuple(ms_init))


def kernel(predictions):
    b = predictions.shape[0]
    pred = jnp.pad(predictions, ((0, 0), (0, _NPAD - _N), (0, 0)))
    pred = pred.transpose(0, 2, 1).reshape(b, 4 + _NCLS, _ROWS, _LANES)
    out = pl.pallas_call(
        _nms_kernel,
        out_shape=jax.ShapeDtypeStruct((b, _MAXDET, _LANES), jnp.float32),
        scratch_shapes=[pltpu.VMEM((_B, _ROWS, _LANES), jnp.float32)] * 6,
    )(pred)
    return out[:, :, :6]
